# Initial kernel scaffold; baseline (speedup 1.0000x reference)
import jax
import jax.numpy as jnp
from jax.experimental import pallas as pl


def kernel(x_embeddings, edge_index, edge_features, batch, Wm0, bm0, Wu0, bu0, Wm1, bm1, Wu1, bu1, Wm2, bm2, Wu2, bu2, Wout, bout):
    raise NotImplementedError("write your pallas kernel here")


# speedup vs baseline: 1.7951x; 1.7951x over previous
import functools

import jax
import jax.numpy as jnp
from jax import lax
from jax.experimental import pallas as pl
from jax.experimental.pallas import tpu as pltpu
from jax.experimental.pallas import tpu_sc as plsc

_N = 10000
_E = 160000
_H = 256
_HH = 128
_G = 64
_NS = 16
_CHUNK = 80
_EPT = _E // _NS
_NCHUNK = _EPT // _CHUNK
_NPAD = 10240
_RPT = _NPAD // _NS
_BN = 400
_NB = _N // _BN
_BU = 80
_NBU = _N // _BU
_BE = 2000
_NEB = _E // _BE

_f32 = jnp.float32


def _proj_body(h_ref, w_ref, out_ref):
    out_ref[...] = jnp.dot(h_ref[...], w_ref[...], preferred_element_type=_f32)


def _proj(h, Wm):
    return pl.pallas_call(
        _proj_body,
        grid=(4, _NB),
        in_specs=[
            pl.BlockSpec((_BN, _H), lambda b, i: (i, 0)),
            pl.BlockSpec((_H, _HH), lambda b, i: (b % 2, b // 2)),
        ],
        out_specs=pl.BlockSpec((_BN, _HH), lambda b, i: (b * _NB + i, 0)),
        out_shape=jax.ShapeDtypeStruct((4 * _N, _HH), _f32),
    )(h, Wm)


def _q_body(e_ref, w_ref, b_ref, out_ref):
    out_ref[...] = jnp.dot(e_ref[...], w_ref[...],
                           preferred_element_type=_f32) + b_ref[...]


def _qproj(e, Wm, bm2d):
    return pl.pallas_call(
        _q_body,
        grid=(2, _NEB),
        in_specs=[
            pl.BlockSpec((_BE, 16), lambda c, i: (i, 0)),
            pl.BlockSpec((16, _HH), lambda c, i: (2 * _H // 16, c)),
            pl.BlockSpec((1, _HH), lambda c, i: (0, c)),
        ],
        out_specs=pl.BlockSpec((_BE, _HH), lambda c, i: (c * _NEB + i, 0)),
        out_shape=jax.ShapeDtypeStruct((2 * _E, _HH), _f32),
    )(e, Wm, bm2d)


def _update_body(h_ref, alo_ref, ahi_ref, dlo_ref, dhi_ref, wu_ref, bu_ref,
                 out_ref):
    deg = dlo_ref[:, 0:1] + dhi_ref[:, 0:1]
    inv = 1.0 / jnp.maximum(deg, 1.0)
    acc = jnp.dot(h_ref[...], wu_ref[0:_H, :], preferred_element_type=_f32)
    acc += jnp.dot(alo_ref[...] * inv, wu_ref[_H:_H + _HH, :],
                   preferred_element_type=_f32)
    acc += jnp.dot(ahi_ref[...] * inv, wu_ref[_H + _HH:2 * _H, :],
                   preferred_element_type=_f32)
    out_ref[...] = jnp.maximum(acc + bu_ref[...], 0.0)


def _update(h, agg_cat, deg, Wu, bu2d):
    return pl.pallas_call(
        _update_body,
        grid=(_NBU,),
        in_specs=[
            pl.BlockSpec((_BU, _H), lambda i: (i, 0)),
            pl.BlockSpec((_BU, _HH), lambda i: (i, 0)),
            pl.BlockSpec((_BU, _HH), lambda i: (_NPAD // _BU + i, 0)),
            pl.BlockSpec((_BU, _HH), lambda i: (i, 0)),
            pl.BlockSpec((_BU, _HH), lambda i: (_NPAD // _BU + i, 0)),
            pl.BlockSpec((2 * _H, _H), lambda i: (0, 0)),
            pl.BlockSpec((1, _H), lambda i: (0, 0)),
        ],
        out_specs=pl.BlockSpec((_BU, _H), lambda i: (i, 0)),
        out_shape=jax.ShapeDtypeStruct((_N, _H), _f32),
    )(h, agg_cat, agg_cat, deg, deg, Wu, bu2d)


def _pool_body(h_ref, b_ref, wout_ref, bout_ref, out_ref):
    seg = lax.broadcasted_iota(jnp.int32, (_G, _N), 0)
    oh = (seg == b_ref[...]).astype(_f32)
    sums = jnp.dot(oh, h_ref[...], preferred_element_type=_f32)
    cnt = jnp.sum(oh, axis=1, keepdims=True)
    ge = sums / jnp.maximum(cnt, 1.0)
    out_ref[...] = jnp.dot(ge, wout_ref[...],
                           preferred_element_type=_f32) + bout_ref[...]


def _pool(h, batch_row, wout_pad, bout_pad):
    return pl.pallas_call(
        _pool_body,
        out_shape=jax.ShapeDtypeStruct((_G, _HH), _f32),
    )(h, batch_row, wout_pad, bout_pad)


def _make_sc_edge():
    mesh = plsc.VectorSubcoreMesh(core_axis_name="c", subcore_axis_name="s")

    out_type = [jax.ShapeDtypeStruct((2 * _NPAD, _HH), _f32)]
    scratch = [
        pltpu.VMEM_SHARED((_NPAD, _HH), _f32),
        pltpu.VMEM((_CHUNK,), jnp.int32),
        pltpu.VMEM((_CHUNK,), jnp.int32),
        pltpu.VMEM((_CHUNK,), jnp.int32),
        pltpu.VMEM((_CHUNK,), jnp.int32),
        pltpu.VMEM((_CHUNK, _HH), _f32),
        pltpu.VMEM((_CHUNK, _HH), _f32),
        pltpu.SemaphoreType.DMA,
        pltpu.SemaphoreType.DMA,
    ]

    def body(src_hbm, dst_hbm, t_hbm, q_hbm, agg_out, *rest):
        (agg_sh, src_v, dst_v, sidx_v, didx_v, ps_v, q_v,
         sem0, sem1) = rest

        c = lax.axis_index("c")
        s = lax.axis_index("s")
        zv = jnp.zeros((16,), _f32)

        def zrow(r, _):
            for k in range(8):
                ps_v[r, pl.ds(k * 16, 16)] = zv
            return 0
        lax.fori_loop(0, _CHUNK, zrow, 0)

        base_r = s * _RPT

        def zcp(i, _):
            pltpu.sync_copy(ps_v, agg_sh.at[pl.ds(base_r + i * _CHUNK, _CHUNK)])
            return 0
        lax.fori_loop(0, _RPT // _CHUNK, zcp, 0)

        plsc.subcore_barrier()

        ebase = s * _EPT
        soff = (2 * c) * _N
        doff = soff + _N
        qbase = c * _E + ebase

        def chunk_body(ch, _):
            eb = ebase + ch * _CHUNK
            pltpu.sync_copy(src_hbm.at[pl.ds(eb, _CHUNK)], src_v)
            pltpu.sync_copy(dst_hbm.at[pl.ds(eb, _CHUNK)], dst_v)
            for k in range(_CHUNK // 16):
                sl = pl.ds(k * 16, 16)
                sidx_v[sl] = src_v[sl] + soff
                didx_v[sl] = dst_v[sl] + doff
            cp1 = pltpu.async_copy(t_hbm.at[sidx_v], ps_v, sem0)
            pltpu.sync_copy(q_hbm.at[pl.ds(qbase + ch * _CHUNK, _CHUNK)], q_v)
            cp1.wait()

            def arow(r, _):
                for k in range(8):
                    sl = pl.ds(k * 16, 16)
                    ps_v[r, sl] = ps_v[r, sl] + q_v[r, sl]
                return 0
            lax.fori_loop(0, _CHUNK, arow, 0)

            cp2 = pltpu.async_copy(t_hbm.at[didx_v], q_v, sem1)
            cp2.wait()

            def mrow(r, _):
                for k in range(8):
                    sl = pl.ds(k * 16, 16)
                    ps_v[r, sl] = jnp.maximum(ps_v[r, sl] + q_v[r, sl], 0.0)
                return 0
            lax.fori_loop(0, _CHUNK, mrow, 0)

            pltpu.sync_copy(ps_v, agg_sh.at[dst_v], add=True)
            return 0
        lax.fori_loop(0, _NCHUNK, chunk_body, 0)

        plsc.subcore_barrier()

        pltpu.sync_copy(agg_sh.at[pl.ds(base_r, _RPT)],
                        agg_out.at[pl.ds(c * _NPAD + base_r, _RPT)])

    return functools.partial(
        pl.kernel, mesh=mesh, out_type=out_type, scratch_types=scratch)(body)


_sc_edge = _make_sc_edge()

_DCH = 200
_DEPT = _E // (2 * _NS)
_NDCH = _DEPT // _DCH


def _make_sc_deg():
    mesh = plsc.VectorSubcoreMesh(core_axis_name="c", subcore_axis_name="s")

    out_type = [jax.ShapeDtypeStruct((2 * _NPAD, _HH), _f32)]
    scratch = [
        pltpu.VMEM_SHARED((_NPAD, _HH), _f32),
        pltpu.VMEM((_DCH,), jnp.int32),
        pltpu.VMEM((_DCH, _HH), _f32),
    ]

    def body(dst_hbm, deg_out, deg_sh, dst_v, ones_v):
        c = lax.axis_index("c")
        s = lax.axis_index("s")
        zv = jnp.zeros((16,), _f32)

        def zrow(r, _):
            for k in range(8):
                ones_v[r, pl.ds(k * 16, 16)] = zv
            return 0
        lax.fori_loop(0, _DCH, zrow, 0)

        base_r = s * _RPT

        def zcp(i, _):
            pltpu.sync_copy(ones_v.at[pl.ds(0, 160)],
                            deg_sh.at[pl.ds(base_r + i * 160, 160)])
            return 0
        lax.fori_loop(0, _RPT // 160, zcp, 0)

        ov = jnp.ones((16,), _f32)

        def orow(r, _):
            for k in range(8):
                ones_v[r, pl.ds(k * 16, 16)] = ov
            return 0
        lax.fori_loop(0, _DCH, orow, 0)

        plsc.subcore_barrier()

        ebase = c * (_E // 2) + s * _DEPT

        def chunk_body(ch, _):
            pltpu.sync_copy(dst_hbm.at[pl.ds(ebase + ch * _DCH, _DCH)], dst_v)
            pltpu.sync_copy(ones_v, deg_sh.at[dst_v], add=True)
            return 0
        lax.fori_loop(0, _NDCH, chunk_body, 0)

        plsc.subcore_barrier()

        pltpu.sync_copy(deg_sh.at[pl.ds(base_r, _RPT)],
                        deg_out.at[pl.ds(c * _NPAD + base_r, _RPT)])

    return functools.partial(
        pl.kernel, mesh=mesh, out_type=out_type, scratch_types=scratch)(body)


_sc_deg = _make_sc_deg()


def kernel(x_embeddings, edge_index, edge_features, batch,
           Wm0, bm0, Wu0, bu0, Wm1, bm1, Wu1, bu1, Wm2, bm2, Wu2, bu2,
           Wout, bout):
    src = edge_index[0].astype(jnp.int32)
    dst = edge_index[1].astype(jnp.int32)
    batch_row = batch.astype(jnp.int32).reshape(1, _N)
    wout_pad = jnp.pad(Wout, ((0, 0), (0, _HH - 2)))
    bout_pad = jnp.pad(bout.reshape(1, 2), ((0, 0), (0, _HH - 2)))

    h = x_embeddings
    (deg,) = _sc_deg(dst)
    for Wm, bm, Wu, bu in [(Wm0, bm0, Wu0, bu0),
                           (Wm1, bm1, Wu1, bu1),
                           (Wm2, bm2, Wu2, bu2)]:
        T = _proj(h, Wm)
        Q = _qproj(edge_features, Wm, bm.reshape(1, _H))
        (agg_cat,) = _sc_edge(src, dst, T, Q)
        h = _update(h, agg_cat, deg, Wu, bu.reshape(1, _H))

    out = _pool(h, batch_row, wout_pad, bout_pad)
    return out[:, :2]


# speedup vs baseline: 2.0575x; 1.1462x over previous
import functools

import jax
import jax.numpy as jnp
from jax import lax
from jax.experimental import pallas as pl
from jax.experimental.pallas import tpu as pltpu
from jax.experimental.pallas import tpu_sc as plsc

_N = 10000
_E = 160000
_H = 256
_HH = 128
_G = 64
_NS = 16
_CHUNK = 80
_EPT = _E // _NS
_NCHUNK = _EPT // _CHUNK
_NPAD = 10240
_RPT = _NPAD // _NS
_BN = 400
_NB = _N // _BN
_BU = 80
_NBU = _N // _BU
_BE = 2000
_NEB = _E // _BE

_f32 = jnp.float32


def _proj_body(h_ref, w_ref, out_ref):
    out_ref[...] = jnp.dot(h_ref[...], w_ref[...], preferred_element_type=_f32)


def _proj(h, Wm):
    return pl.pallas_call(
        _proj_body,
        grid=(4, _NB),
        in_specs=[
            pl.BlockSpec((_BN, _H), lambda b, i: (i, 0)),
            pl.BlockSpec((_H, _HH), lambda b, i: (b % 2, b // 2)),
        ],
        out_specs=pl.BlockSpec((_BN, _HH), lambda b, i: (b * _NB + i, 0)),
        out_shape=jax.ShapeDtypeStruct((4 * _N, _HH), _f32),
    )(h, Wm)


def _q_body(e_ref, w_ref, b_ref, out_ref):
    out_ref[...] = jnp.dot(e_ref[...], w_ref[...],
                           preferred_element_type=_f32) + b_ref[...]


def _qproj(e, Wm, bm2d):
    return pl.pallas_call(
        _q_body,
        grid=(2, _NEB),
        in_specs=[
            pl.BlockSpec((_BE, 16), lambda c, i: (i, 0)),
            pl.BlockSpec((16, _HH), lambda c, i: (2 * _H // 16, c)),
            pl.BlockSpec((1, _HH), lambda c, i: (0, c)),
        ],
        out_specs=pl.BlockSpec((_BE, _HH), lambda c, i: (c * _NEB + i, 0)),
        out_shape=jax.ShapeDtypeStruct((2 * _E, _HH), _f32),
    )(e, Wm, bm2d)


def _update_body(h_ref, alo_ref, ahi_ref, dlo_ref, dhi_ref, wu_ref, bu_ref,
                 out_ref):
    deg = dlo_ref[:, 0:1] + dhi_ref[:, 0:1]
    inv = 1.0 / jnp.maximum(deg, 1.0)
    acc = jnp.dot(h_ref[...], wu_ref[0:_H, :], preferred_element_type=_f32)
    acc += jnp.dot(alo_ref[...] * inv, wu_ref[_H:_H + _HH, :],
                   preferred_element_type=_f32)
    acc += jnp.dot(ahi_ref[...] * inv, wu_ref[_H + _HH:2 * _H, :],
                   preferred_element_type=_f32)
    out_ref[...] = jnp.maximum(acc + bu_ref[...], 0.0)


def _update(h, agg_cat, deg, Wu, bu2d):
    return pl.pallas_call(
        _update_body,
        grid=(_NBU,),
        in_specs=[
            pl.BlockSpec((_BU, _H), lambda i: (i, 0)),
            pl.BlockSpec((_BU, _HH), lambda i: (i, 0)),
            pl.BlockSpec((_BU, _HH), lambda i: (_NPAD // _BU + i, 0)),
            pl.BlockSpec((_BU, _HH), lambda i: (i, 0)),
            pl.BlockSpec((_BU, _HH), lambda i: (_NPAD // _BU + i, 0)),
            pl.BlockSpec((2 * _H, _H), lambda i: (0, 0)),
            pl.BlockSpec((1, _H), lambda i: (0, 0)),
        ],
        out_specs=pl.BlockSpec((_BU, _H), lambda i: (i, 0)),
        out_shape=jax.ShapeDtypeStruct((_N, _H), _f32),
    )(h, agg_cat, agg_cat, deg, deg, Wu, bu2d)


def _pool_body(h_ref, b_ref, wout_ref, bout_ref, out_ref):
    seg = lax.broadcasted_iota(jnp.int32, (_G, _N), 0)
    oh = (seg == b_ref[...]).astype(_f32)
    sums = jnp.dot(oh, h_ref[...], preferred_element_type=_f32)
    cnt = jnp.sum(oh, axis=1, keepdims=True)
    ge = sums / jnp.maximum(cnt, 1.0)
    out_ref[...] = jnp.dot(ge, wout_ref[...],
                           preferred_element_type=_f32) + bout_ref[...]


def _pool(h, batch_row, wout_pad, bout_pad):
    return pl.pallas_call(
        _pool_body,
        out_shape=jax.ShapeDtypeStruct((_G, _HH), _f32),
    )(h, batch_row, wout_pad, bout_pad)


def _make_sc_edge():
    mesh = plsc.VectorSubcoreMesh(core_axis_name="c", subcore_axis_name="s")

    out_type = [jax.ShapeDtypeStruct((2 * _NPAD, _HH), _f32)]
    scratch = [
        pltpu.VMEM_SHARED((_NPAD, _HH), _f32),
        pltpu.VMEM((_CHUNK,), jnp.int32),
        pltpu.VMEM((_CHUNK,), jnp.int32),
        pltpu.VMEM((_CHUNK,), jnp.int32),
        pltpu.VMEM((_CHUNK,), jnp.int32),
        pltpu.VMEM((_CHUNK, _HH), _f32),
        pltpu.VMEM((_CHUNK, _HH), _f32),
        pltpu.VMEM((_CHUNK, _HH), _f32),
        pltpu.SemaphoreType.DMA,
        pltpu.SemaphoreType.DMA,
    ]

    def body(src_hbm, dst_hbm, t_hbm, q_hbm, agg_out, *rest):
        (agg_sh, src_v, dst_v, sidx_v, didx_v, ps_v, pd_v, q_v,
         sem0, sem1) = rest

        c = lax.axis_index("c")
        s = lax.axis_index("s")
        zv = jnp.zeros((16,), _f32)

        def zrow(r, _):
            for k in range(8):
                ps_v[r, pl.ds(k * 16, 16)] = zv
            return 0
        lax.fori_loop(0, _CHUNK, zrow, 0)

        base_r = s * _RPT

        def zcp(i, _):
            pltpu.sync_copy(ps_v, agg_sh.at[pl.ds(base_r + i * _CHUNK, _CHUNK)])
            return 0
        lax.fori_loop(0, _RPT // _CHUNK, zcp, 0)

        plsc.subcore_barrier()

        ebase = s * _EPT
        soff = (2 * c) * _N
        doff = soff + _N
        qbase = c * _E + ebase

        def chunk_body(ch, _):
            eb = ebase + ch * _CHUNK
            pltpu.sync_copy(src_hbm.at[pl.ds(eb, _CHUNK)], src_v)
            pltpu.sync_copy(dst_hbm.at[pl.ds(eb, _CHUNK)], dst_v)
            for k in range(_CHUNK // 16):
                sl = pl.ds(k * 16, 16)
                sidx_v[sl] = src_v[sl] + soff
                didx_v[sl] = dst_v[sl] + doff
            cp1 = pltpu.async_copy(t_hbm.at[sidx_v], ps_v, sem0)
            cp2 = pltpu.async_copy(t_hbm.at[didx_v], pd_v, sem1)
            pltpu.sync_copy(q_hbm.at[pl.ds(qbase + ch * _CHUNK, _CHUNK)], q_v)
            cp1.wait()
            cp2.wait()

            def mrow(r2, _):
                for dr in range(2):
                    r = r2 * 2 + dr
                    for k in range(8):
                        sl = pl.ds(k * 16, 16)
                        ps_v[r, sl] = jnp.maximum(
                            ps_v[r, sl] + pd_v[r, sl] + q_v[r, sl], 0.0)
                return 0
            lax.fori_loop(0, _CHUNK // 2, mrow, 0)

            pltpu.sync_copy(ps_v, agg_sh.at[dst_v], add=True)
            return 0
        lax.fori_loop(0, _NCHUNK, chunk_body, 0)

        plsc.subcore_barrier()

        pltpu.sync_copy(agg_sh.at[pl.ds(base_r, _RPT)],
                        agg_out.at[pl.ds(c * _NPAD + base_r, _RPT)])

    return functools.partial(
        pl.kernel, mesh=mesh, out_type=out_type, scratch_types=scratch)(body)


_sc_edge = _make_sc_edge()

_DCH = 200
_DEPT = _E // (2 * _NS)
_NDCH = _DEPT // _DCH


def _make_sc_deg():
    mesh = plsc.VectorSubcoreMesh(core_axis_name="c", subcore_axis_name="s")

    out_type = [jax.ShapeDtypeStruct((2 * _NPAD, _HH), _f32)]
    scratch = [
        pltpu.VMEM_SHARED((_NPAD, _HH), _f32),
        pltpu.VMEM((_DCH,), jnp.int32),
        pltpu.VMEM((_DCH, _HH), _f32),
    ]

    def body(dst_hbm, deg_out, deg_sh, dst_v, ones_v):
        c = lax.axis_index("c")
        s = lax.axis_index("s")
        zv = jnp.zeros((16,), _f32)

        def zrow(r, _):
            for k in range(8):
                ones_v[r, pl.ds(k * 16, 16)] = zv
            return 0
        lax.fori_loop(0, _DCH, zrow, 0)

        base_r = s * _RPT

        def zcp(i, _):
            pltpu.sync_copy(ones_v.at[pl.ds(0, 160)],
                            deg_sh.at[pl.ds(base_r + i * 160, 160)])
            return 0
        lax.fori_loop(0, _RPT // 160, zcp, 0)

        ov = jnp.ones((16,), _f32)

        def orow(r, _):
            for k in range(8):
                ones_v[r, pl.ds(k * 16, 16)] = ov
            return 0
        lax.fori_loop(0, _DCH, orow, 0)

        plsc.subcore_barrier()

        ebase = c * (_E // 2) + s * _DEPT

        def chunk_body(ch, _):
            pltpu.sync_copy(dst_hbm.at[pl.ds(ebase + ch * _DCH, _DCH)], dst_v)
            pltpu.sync_copy(ones_v, deg_sh.at[dst_v], add=True)
            return 0
        lax.fori_loop(0, _NDCH, chunk_body, 0)

        plsc.subcore_barrier()

        pltpu.sync_copy(deg_sh.at[pl.ds(base_r, _RPT)],
                        deg_out.at[pl.ds(c * _NPAD + base_r, _RPT)])

    return functools.partial(
        pl.kernel, mesh=mesh, out_type=out_type, scratch_types=scratch)(body)


_sc_deg = _make_sc_deg()


def kernel(x_embeddings, edge_index, edge_features, batch,
           Wm0, bm0, Wu0, bu0, Wm1, bm1, Wu1, bu1, Wm2, bm2, Wu2, bu2,
           Wout, bout):
    src = edge_index[0].astype(jnp.int32)
    dst = edge_index[1].astype(jnp.int32)
    batch_row = batch.astype(jnp.int32).reshape(1, _N)
    wout_pad = jnp.pad(Wout, ((0, 0), (0, _HH - 2)))
    bout_pad = jnp.pad(bout.reshape(1, 2), ((0, 0), (0, _HH - 2)))

    h = x_embeddings
    (deg,) = _sc_deg(dst)
    for Wm, bm, Wu, bu in [(Wm0, bm0, Wu0, bu0),
                           (Wm1, bm1, Wu1, bu1),
                           (Wm2, bm2, Wu2, bu2)]:
        T = _proj(h, Wm)
        Q = _qproj(edge_features, Wm, bm.reshape(1, _H))
        (agg_cat,) = _sc_edge(src, dst, T, Q)
        h = _update(h, agg_cat, deg, Wu, bu.reshape(1, _H))

    out = _pool(h, batch_row, wout_pad, bout_pad)
    return out[:, :2]
